# baseline (speedup 1.0000x reference)
import functools

import jax
import jax.numpy as jnp
from jax import lax
from jax.experimental import pallas as pl
from jax.experimental.pallas import tpu as pltpu
from jax.experimental.pallas import tpu_sc as plsc
from jax._src.pallas import mpmd as _plmpmd

NC, NS = 2, 16
NW = NC * NS
CH = 40
BM = 1024


def _proj_body(x_ref, w_ref, b_ref, o_ref):
    o_ref[...] = (
        jnp.dot(x_ref[...], w_ref[...], preferred_element_type=jnp.float32)
        + b_ref[...]
    )


def _mask_body(m_ref, o_ref):
    o_ref[...] = m_ref[...] == 0


def _project(emb_table, W, b):
    vocab, lang_dim = emb_table.shape
    out_dim = W.shape[1]
    return pl.pallas_call(
        _proj_body,
        grid=(pl.cdiv(vocab, BM),),
        in_specs=[
            pl.BlockSpec((BM, lang_dim), lambda i: (i, 0)),
            pl.BlockSpec((lang_dim, out_dim), lambda i: (0, 0)),
            pl.BlockSpec((1, out_dim), lambda i: (0, 0)),
        ],
        out_specs=pl.BlockSpec((BM, out_dim), lambda i: (i, 0)),
        out_shape=jax.ShapeDtypeStruct((vocab, out_dim), jnp.float32),
    )(emb_table, W, b.reshape(1, out_dim))


@functools.cache
def _gather_half_call(total_rows, out_dim, half, c0, with_prev):
    n_per_w = total_rows // NW
    n_chunks = n_per_w // CH
    n_pairs = n_chunks // 2
    mesh = plsc.VectorSubcoreMesh(core_axis_name="c", subcore_axis_name="s")

    def gk(*refs):
        if with_prev:
            tbl_hbm, idx_hbm, _prev, out_hbm = refs[:4]
            scr = refs[4:]
        else:
            tbl_hbm, idx_hbm, out_hbm = refs[:3]
            scr = refs[3:]
        idx_v, buf0, buf1, gs0, gs1, os0, os1 = scr

        wid = lax.axis_index("s") * NC + lax.axis_index("c")
        base = wid * n_per_w
        pltpu.sync_copy(idx_hbm.at[wid], idx_v)

        def gather(a, buf, sem):
            return pltpu.make_async_copy(tbl_hbm.at[idx_v.at[a]], buf, sem)

        def writeback(a, buf, sem):
            return pltpu.make_async_copy(
                buf, out_hbm.at[pl.ds(base + a * CH, CH), pl.ds(c0, half)], sem)

        gather(0, buf0, gs0).start()

        def body(g, carry):
            a = 2 * g

            @pl.when(g > 0)
            def _():
                writeback(a - 1, buf1, os1).wait()

            gather(a + 1, buf1, gs1).start()
            gather(a, buf0, gs0).wait()
            writeback(a, buf0, os0).start()

            @pl.when(g + 1 < n_pairs)
            def _():
                writeback(a, buf0, os0).wait()
                gather(a + 2, buf0, gs0).start()

            gather(a + 1, buf1, gs1).wait()
            writeback(a + 1, buf1, os1).start()
            return carry

        lax.fori_loop(0, n_pairs, body, 0)
        writeback(n_chunks - 2, buf0, os0).wait()
        writeback(n_chunks - 1, buf1, os1).wait()

    return _plmpmd._mpmd_map(
        [(mesh, gk)],
        out_types=jax.ShapeDtypeStruct((total_rows, out_dim), jnp.float32),
        input_output_aliases={2: 0} if with_prev else {},
        scratch_types=[
            pltpu.VMEM((n_chunks, CH), jnp.int32),
            pltpu.VMEM((CH, half), jnp.float32),
            pltpu.VMEM((CH, half), jnp.float32),
            pltpu.SemaphoreType.DMA,
            pltpu.SemaphoreType.DMA,
            pltpu.SemaphoreType.DMA,
            pltpu.SemaphoreType.DMA,
        ],
    )


def kernel(ref_expr_inds, attention_mask, emb_table, W, b):
    B, S = ref_expr_inds.shape
    out_dim = W.shape[1]
    total = B * S
    half = out_dim // 2

    proj0 = _project(emb_table, W[:, :half], b[:half])
    proj1 = _project(emb_table, W[:, half:], b[half:])

    idx3 = ref_expr_inds.T.reshape(NW, total // NW // CH, CH)
    g0 = _gather_half_call(total, out_dim, half, 0, False)(proj0, idx3)
    g1 = _gather_half_call(total, out_dim, half, half, True)(proj1, idx3, g0)
    y = g1.reshape(S, B, out_dim).transpose(1, 0, 2)

    pad_mask = pl.pallas_call(
        _mask_body,
        out_shape=jax.ShapeDtypeStruct((B, S), jnp.bool_),
    )(attention_mask)
    return (y, pad_mask)
